# baseline (speedup 1.0000x reference)
import jax
import jax.numpy as jnp
from jax.experimental import pallas as pl
from jax.experimental.pallas import tpu as pltpu

_LATENTS = (32, 64, 96, 128, 192, 256)
_NH = 6
_DIN = 73
_DMAX = 256
_TILE = 512


def _fused_body(eq_ref, csi_ref, np_ref, r_ref, *refs):
    o_ref = refs[-8]
    w1c_s, b1c_s, a1c_s, w3c_s, bc_s, lnb_s, aux_s = refs[-7:]
    pr = refs[:-8]

    pid = pl.program_id(0)

    @pl.when(pid == 0)
    def _prep():
        for i in range(_NH):
            (w1_r, b1_r, a1_r, _w2, _b2, _a2, w3_r, b3_r, lnw_r, lnb_r) = \
                pr[10 * i:10 * (i + 1)]
            d = _LATENTS[i]
            w1c_s[i * 128:(i + 1) * 128, :] = w1_r[...]
            b1c_s[0, i * 128:(i + 1) * 128] = b1_r[...]
            a1c_s[0, i * 128:(i + 1) * 128] = jnp.full((128,), a1_r[0],
                                                       jnp.float32)
            w3 = w3_r[...]
            wbar = jnp.mean(w3, axis=0, keepdims=True)
            w3p = w3 - wbar
            lnw = lnw_r[...]
            b3 = b3_r[...]
            b3p = b3 - jnp.mean(b3)
            z2 = jnp.zeros((_DMAX - d, 64), jnp.float32) if d < _DMAX else None
            sc = w3p * lnw[:, None]
            if z2 is not None:
                w3c_s[i] = jnp.concatenate([sc, z2, w3p, z2], axis=0)
                zp = jnp.zeros((_DMAX - d,), jnp.float32)
                bc_s[i, :] = jnp.concatenate([lnw * b3p, zp, b3p, zp])
                lnb_s[i, :] = jnp.concatenate([lnb_r[...], zp])
            else:
                w3c_s[i] = jnp.concatenate([sc, w3p], axis=0)
                bc_s[i, :] = jnp.concatenate([lnw * b3p, b3p])
                lnb_s[i, :] = lnb_r[...]
            aux_s[i, :] = jnp.full((8,), 1.0 / d, jnp.float32)

    @pl.when(pid > 0)
    def _main():
        r = r_ref[...]
        best = jnp.max(r, axis=1, keepdims=True)
        eqm = (r == best).astype(jnp.float32)
        lt = jnp.tril(jnp.ones((_NH, _NH), jnp.float32))
        cums = jax.lax.dot_general(eqm, lt, (((1,), (0,)), ((), ())),
                                   preferred_element_type=jnp.float32)
        fm = eqm * (cums == 1.0).astype(jnp.float32)

        x = jnp.concatenate([eq_ref[...], csi_ref[...], np_ref[...]], axis=1)
        h1 = jax.lax.dot_general(x, w1c_s[...], (((1,), (1,)), ((), ())),
                                 preferred_element_type=jnp.float32)
        h1 = h1 + b1c_s[...]
        h1 = jnp.where(h1 >= 0, h1, a1c_s[...] * h1)

        acc = jnp.zeros((x.shape[0], 2 * _DMAX), jnp.float32)
        for i in range(_NH):
            w2_r, b2_r, a2_r = pr[10 * i + 3:10 * i + 6]
            h = h1[:, i * 128:(i + 1) * 128]
            h2 = jax.lax.dot_general(h, w2_r[...], (((1,), (1,)), ((), ())),
                                     preferred_element_type=jnp.float32)
            h2 = h2 + b2_r[...]
            h2 = jnp.where(h2 >= 0, h2, a2_r[0] * h2)
            h2 = h2 * fm[:, i:i + 1]
            acc = acc + jax.lax.dot_general(
                h2, w3c_s[i], (((1,), (1,)), ((), ())),
                preferred_element_type=jnp.float32)
        acc = acc + jax.lax.dot_general(fm, bc_s[...], (((1,), (0,)), ((), ())),
                                        preferred_element_type=jnp.float32)
        z = acc[:, :_DMAX]
        u = acc[:, _DMAX:]
        ss = jax.lax.dot_general(u * u, jnp.ones((1, _DMAX), jnp.float32),
                                 (((1,), (1,)), ((), ())),
                                 preferred_element_type=jnp.float32)
        invd = jax.lax.dot_general(fm, aux_s[...], (((1,), (0,)), ((), ())),
                                   preferred_element_type=jnp.float32)
        lnb = jax.lax.dot_general(fm, lnb_s[...], (((1,), (0,)), ((), ())),
                                  preferred_element_type=jnp.float32)
        rs = jax.lax.rsqrt(ss * invd[:, 0:1] + 1e-5)
        o_ref[...] = z * rs + lnb


def kernel(equalized_symbol, csi_context, noise_power, rate_one_hot, params):
    b = equalized_symbol.shape[0]
    nt = b // _TILE
    grid = (nt + 1,)
    tb = lambda w: pl.BlockSpec((_TILE, w),
                                lambda i: (jnp.maximum(i - 1, 0), 0))
    full = lambda a: pl.BlockSpec(a.shape, lambda i: (0,) * a.ndim)
    smem = pl.BlockSpec(memory_space=pltpu.SMEM)

    pargs, pspecs = [], []
    for p in params:
        for k in ('W1', 'b1', 'a1', 'W2', 'b2', 'a2', 'W3', 'b3', 'ln_w', 'ln_b'):
            v = p[k]
            pargs.append(v)
            pspecs.append(smem if k in ('a1', 'a2') else full(v))

    out = pl.pallas_call(
        _fused_body,
        grid=grid,
        in_specs=[tb(8), tb(64), tb(1), tb(_NH), *pspecs],
        out_specs=pl.BlockSpec((_TILE, _DMAX),
                               lambda i: (jnp.maximum(i - 1, 0), 0)),
        out_shape=jax.ShapeDtypeStruct((b, _DMAX), jnp.float32),
        scratch_shapes=[
            pltpu.VMEM((_NH * 128, _DIN), jnp.float32),
            pltpu.VMEM((1, _NH * 128), jnp.float32),
            pltpu.VMEM((1, _NH * 128), jnp.float32),
            pltpu.VMEM((_NH, 2 * _DMAX, 64), jnp.float32),
            pltpu.VMEM((_NH, 2 * _DMAX), jnp.float32),
            pltpu.VMEM((_NH, _DMAX), jnp.float32),
            pltpu.VMEM((_NH, 8), jnp.float32),
        ],
    )(equalized_symbol, csi_context, noise_power[:, None], rate_one_hot,
      *pargs)
    return out
